# Initial kernel scaffold; baseline (speedup 1.0000x reference)
import jax
import jax.numpy as jnp
from jax.experimental import pallas as pl


def kernel(input_ids, segment_ids, token_table, pos_table, seg_table, gamma, beta):
    raise NotImplementedError("write your pallas kernel here")


# speedup vs baseline: 5.6194x; 5.6194x over previous
import functools

import jax
import jax.numpy as jnp
from jax import lax
from jax.experimental import pallas as pl
from jax.experimental.pallas import tpu as pltpu
from jax.experimental.pallas import tpu_sc as plsc

NC = 2
NS = 16
NW = NC * NS

VOCAB = 100000
EMBED = 128
MAX_POS = 128
NUM_SEG = 2
B = 1024
L = 128

SEQ_PER_W = B // NW
NCHUNK = SEQ_PER_W
CSZ = L


def _gather_body(ids_hbm, tok_hbm, out_hbm, idx_v, buf_v, sem):
    w = lax.axis_index("s") * NC + lax.axis_index("c")
    pltpu.sync_copy(ids_hbm.at[w], idx_v)

    pltpu.async_copy(tok_hbm.at[idx_v.at[0]], buf_v.at[0], sem)

    def chunk(g, carry):
        cur = lax.rem(g, 2)
        nxt = 1 - cur

        @pl.when(g + 1 < NCHUNK)
        def _():
            pltpu.async_copy(tok_hbm.at[idx_v.at[g + 1]], buf_v.at[nxt], sem)

        pltpu.make_async_copy(tok_hbm.at[idx_v.at[g]], buf_v.at[cur],
                              sem).wait()
        pltpu.sync_copy(buf_v.at[cur], out_hbm.at[w, g])
        return carry

    lax.fori_loop(0, NCHUNK, chunk, 0)


@functools.partial(jax.jit, static_argnames=())
def _sc_gather(ids, token_table):
    mesh = plsc.VectorSubcoreMesh(core_axis_name="c", subcore_axis_name="s")
    f = pl.kernel(
        _gather_body,
        out_type=jax.ShapeDtypeStruct((NW, NCHUNK, CSZ, EMBED), jnp.float32),
        mesh=mesh,
        scratch_types=[
            pltpu.VMEM((NCHUNK, CSZ), jnp.int32),
            pltpu.VMEM((2, CSZ, EMBED), jnp.float32),
            pltpu.SemaphoreType.DMA,
        ],
    )
    return f(ids, token_table)


S = 16


def _ln_body(x_ref, seg_ref, pos_ref, segtab_ref, gam_ref, bet_ref, o_ref):
    x = x_ref[...]
    x = x + pos_ref[...][None]
    segf = seg_ref[...]
    s0 = segtab_ref[0][None, None]
    s1 = segtab_ref[1][None, None]
    x = x + (s0 + segf * (s1 - s0))
    mean = jnp.mean(x, axis=-1, keepdims=True)
    xc = x - mean
    var = jnp.mean(xc * xc, axis=-1, keepdims=True)
    xhat = xc * lax.rsqrt(var + 1e-5)
    o_ref[...] = xhat * gam_ref[...][None] + bet_ref[...][None]


@jax.jit
def _tc_ln(rows, segs, pos_table, seg_table, gamma, beta):
    grid = (B // S,)
    return pl.pallas_call(
        _ln_body,
        grid=grid,
        in_specs=[
            pl.BlockSpec((S, L, EMBED), lambda i: (i, 0, 0)),
            pl.BlockSpec((S, L, 1), lambda i: (i, 0, 0)),
            pl.BlockSpec((MAX_POS, EMBED), lambda i: (0, 0)),
            pl.BlockSpec((NUM_SEG, EMBED), lambda i: (0, 0)),
            pl.BlockSpec((1, EMBED), lambda i: (0, 0)),
            pl.BlockSpec((1, EMBED), lambda i: (0, 0)),
        ],
        out_specs=pl.BlockSpec((S, L, EMBED), lambda i: (i, 0, 0)),
        out_shape=jax.ShapeDtypeStruct((B, L, EMBED), jnp.float32),
    )(rows, segs, pos_table, seg_table, gamma, beta)


def kernel(input_ids, segment_ids, token_table, pos_table, seg_table, gamma,
           beta):
    ids = jnp.reshape(input_ids, (NW, NCHUNK, CSZ))
    rows = _sc_gather(ids, token_table)
    rows = jnp.reshape(rows, (B, L, EMBED))
    segf = jnp.reshape(segment_ids, (B, L, 1)).astype(jnp.float32)
    return _tc_ln(rows, segf, pos_table, seg_table,
                  jnp.reshape(gamma, (1, EMBED)), jnp.reshape(beta, (1, EMBED)))
